# baseline (speedup 1.0000x reference)
import jax
import jax.numpy as jnp
from jax import lax
from jax.experimental import pallas as pl
from jax.experimental.pallas import tpu as pltpu
from jax.experimental.pallas import tpu_sc as plsc

B, C, HID = 4, 256, 128
H_BEV = W_BEV = 180
HW = H_BEV * W_BEV
NUM_FG = 1000
KPAD = 1024
MCAND = 2048
T_A = 4096
NT_A = (HW + T_A - 1) // T_A
NC, NS = 2, 16
NW = NC * NS


def _fg_body(bev_ref, pos_ref, w1_ref, b1_ref, w2_ref, b2_ref,
             logit_ref, ft_ref):
    f = bev_ref[0] + pos_ref[0]
    ft = jnp.transpose(f)
    ft_ref[0] = ft
    h = lax.dot_general(ft, w1_ref[...], (((1,), (1,)), ((), ())))
    h = jnp.maximum(h + b1_ref[...], 0.0)
    lg = lax.dot_general(w2_ref[...], h, (((1,), (1,)), ((), ())))
    logit_ref[...] = (lg + b2_ref[...])[None]


def _fg_pass(bev3, pos3, fg_w1, fg_b1, fg_w2, fg_b2):
    return pl.pallas_call(
        _fg_body,
        grid=(B, NT_A),
        in_specs=[
            pl.BlockSpec((1, C, T_A), lambda b, t: (b, 0, t)),
            pl.BlockSpec((1, C, T_A), lambda b, t: (b, 0, t)),
            pl.BlockSpec((HID, C), lambda b, t: (0, 0)),
            pl.BlockSpec((1, HID), lambda b, t: (0, 0)),
            pl.BlockSpec((1, HID), lambda b, t: (0, 0)),
            pl.BlockSpec((1, 1), lambda b, t: (0, 0)),
        ],
        out_specs=[
            pl.BlockSpec((1, 1, T_A), lambda b, t: (b, 0, t)),
            pl.BlockSpec((1, T_A, C), lambda b, t: (b, t, 0)),
        ],
        out_shape=[
            jax.ShapeDtypeStruct((B, 1, HW), jnp.float32),
            jax.ShapeDtypeStruct((B, HW, C), jnp.float32),
        ],
    )(bev3, pos3, fg_w1, fg_b1.reshape(1, HID), fg_w2, fg_b2.reshape(1, 1))


def _gather_pass(table, flat_idx):
    nrows = B * KPAD
    b_per_w = nrows // NW
    mesh = plsc.VectorSubcoreMesh(core_axis_name="c", subcore_axis_name="s")

    @pl.kernel(
        mesh=mesh,
        out_type=jax.ShapeDtypeStruct((nrows, C), jnp.float32),
        scratch_types=[
            pltpu.VMEM((b_per_w,), jnp.int32),
            pltpu.VMEM((b_per_w, C), jnp.float32),
            pltpu.SemaphoreType.DMA,
        ],
    )
    def c3(table_hbm, idx_hbm, out_hbm, idx_v, rows_v, sem):
        wid = lax.axis_index("s") * NC + lax.axis_index("c")
        base = wid * b_per_w
        pltpu.sync_copy(idx_hbm.at[pl.ds(base, b_per_w)], idx_v)
        pltpu.async_copy(table_hbm.at[idx_v], rows_v, sem).wait()
        pltpu.sync_copy(rows_v, out_hbm.at[pl.ds(base, b_per_w)])

    return c3(table, flat_idx)


def _head_body(x_ref, idx_ref, qw1_ref, qb1_ref, qw2_ref, qb2_ref,
               pw1_ref, pb1_ref, pw2_ref, qual_ref, qpos_ref):
    x = x_ref[0]
    hq = lax.dot_general(qw1_ref[...], x, (((1,), (1,)), ((), ())))
    hq = jnp.maximum(hq + qb1_ref[...].reshape(HID, 1), 0.0)
    qlg = lax.dot_general(qw2_ref[...], hq, (((1,), (0,)), ((), ())))
    qual_ref[...] = jax.nn.sigmoid(qlg + qb2_ref[...])[None]

    hp = lax.dot_general(pw1_ref[...], x, (((1,), (1,)), ((), ())))
    hp = jnp.maximum(hp + pb1_ref[...].reshape(HID, 1), 0.0)
    po = lax.dot_general(pw2_ref[...], hp, (((1,), (0,)), ((), ())))

    idr = idx_ref[0]
    y_i = lax.div(idr, W_BEV)
    x_i = lax.rem(idr, W_BEV)
    x_n = (x_i.astype(jnp.float32) + 0.5) / W_BEV
    y_n = (y_i.astype(jnp.float32) + 0.5) / H_BEV
    x_w = -51.2 + x_n * (51.2 - (-51.2))
    y_w = -51.2 + y_n * (51.2 - (-51.2))
    z_w = jnp.zeros_like(x_w)
    base = jnp.concatenate([x_w, y_w, z_w], axis=0)
    qpos_ref[...] = (base + jnp.tanh(po) * 5.0)[None]


def _head_pass(feats_sel, sidx, q_w1, q_b1, q_w2, q_b2, p_w1, p_b1, p_w2, p_b2):
    qual3, qpos3 = pl.pallas_call(
        _head_body,
        grid=(B,),
        in_specs=[
            pl.BlockSpec((1, KPAD, C), lambda b: (b, 0, 0)),
            pl.BlockSpec((1, 1, KPAD), lambda b: (b, 0, 0)),
            pl.BlockSpec((HID, C), lambda b: (0, 0)),
            pl.BlockSpec((1, HID), lambda b: (0, 0)),
            pl.BlockSpec((1, HID), lambda b: (0, 0)),
            pl.BlockSpec((1, 1), lambda b: (0, 0)),
            pl.BlockSpec((HID, C), lambda b: (0, 0)),
            pl.BlockSpec((1, HID), lambda b: (0, 0)),
            pl.BlockSpec((3, HID), lambda b: (0, 0)),
        ],
        out_specs=[
            pl.BlockSpec((1, 1, KPAD), lambda b: (b, 0, 0)),
            pl.BlockSpec((1, 3, KPAD), lambda b: (b, 0, 0)),
        ],
        out_shape=[
            jax.ShapeDtypeStruct((B, 1, KPAD), jnp.float32),
            jax.ShapeDtypeStruct((B, 3, KPAD), jnp.float32),
        ],
    )(feats_sel, sidx.reshape(B, 1, KPAD),
      q_w1, q_b1.reshape(1, HID), q_w2, q_b2.reshape(1, 1),
      p_w1, p_b1.reshape(1, HID), p_w2)
    return qual3, qpos3


def kernel(bev_features, pos_embed, fg_w1, fg_b1, fg_w2, fg_b2,
           q_w1, q_b1, q_w2, q_b2, p_w1, p_b1, p_w2, p_b2):
    bev3 = bev_features.reshape(B, C, HW)
    pos3 = pos_embed.reshape(B, C, HW)
    fg_logits3, feats_t = _fg_pass(bev3, pos3, fg_w1, fg_b1, fg_w2, fg_b2)
    fg_logits = fg_logits3.reshape(B, HW)

    fg_probs = jax.nn.sigmoid(fg_logits)
    _, topk_idx = jax.lax.top_k(fg_probs, NUM_FG)

    sidx = jnp.concatenate(
        [topk_idx, jnp.zeros((B, KPAD - NUM_FG), jnp.int32)], axis=1)
    sflat = sidx + jnp.arange(B, dtype=jnp.int32)[:, None] * HW
    gathered = _gather_pass(feats_t.reshape(B * HW, C), sflat.reshape(B * KPAD))
    feats_sel = gathered.reshape(B, KPAD, C)
    qual3, qpos3 = _head_pass(feats_sel, sidx, q_w1, q_b1, q_w2, q_b2,
                              p_w1, p_b1, p_w2, p_b2)

    selected = feats_sel[:, :NUM_FG]
    query_pos = (jnp.swapaxes(qpos3, 1, 2) + p_b2.reshape(1, 1, 3))[:, :NUM_FG]
    quality = qual3[:, 0, :NUM_FG]
    return (selected, query_pos, fg_logits, quality)
